# baseline (speedup 1.0000x reference)
import functools

import jax
import jax.numpy as jnp
from jax import lax
from jax.experimental import pallas as pl
from jax.experimental.pallas import tpu as pltpu
from jax.experimental.pallas import tpu_sc as plsc

E = 1048576
K = 20
B = 2048
NC = 2
NS = 16
NW = NC * NS
EW = E // NW
CH = 512
NCH = EW // CH
NG = CH // 16
NACC = 2 * K + 1

_LOG1P = (2.554673020349618e-07, 0.9999670809438443, -0.49928504912226557,
          0.32722571497202635, -0.22316586411450423, 0.130833427976782,
          -0.05243753706207599, 0.01000928961639147)


def _log1p_poly(u):
    p = jnp.full((16,), _LOG1P[7], jnp.float32)
    for c in _LOG1P[6::-1]:
        p = p * u + c
    return p


def _sc_body(label_h, theta_h, alpha_h, idx_h, idxn_h, idxp_h, out_h,
             acc_v, th_v, al_v, lb_v, ix_v, ixn_v, ixp_v):
    cid = lax.axis_index("c")
    sid = lax.axis_index("s")
    wid = sid * NC + cid
    base = wid * EW

    zero16 = jnp.zeros((16,), jnp.float32)

    def zero_body(i, _):
        acc_v[pl.ds(i * 16, 16)] = zero16
        return _

    lax.fori_loop(0, NACC * B // 16, zero_body, None)

    iota = lax.iota(jnp.int32, 16)
    iotak = iota * K
    lane0 = iota == 0
    lane15 = iota == 15
    cnt_end = (iota + 1).astype(jnp.float32)
    cnt_start = iota.astype(jnp.float32)
    one16 = jnp.full((16,), 1.0, jnp.float32)

    def chunk_body(ci, _):
        e0 = base + ci * CH
        pltpu.sync_copy(theta_h.at[pl.ds(e0 * K, CH * K)], th_v)
        pltpu.sync_copy(alpha_h.at[pl.ds(e0 * K, CH * K)], al_v)
        pltpu.sync_copy(label_h.at[pl.ds(e0, CH)], lb_v)
        pltpu.sync_copy(idx_h.at[pl.ds(e0, CH)], ix_v)
        pltpu.sync_copy(idxn_h.at[pl.ds(e0, CH)], ixn_v)
        pltpu.sync_copy(idxp_h.at[pl.ds(e0, CH)], ixp_v)

        def group_body(g, _):
            o = g * 16
            d = ix_v[pl.ds(o, 16)]
            dn = ixn_v[pl.ds(o, 16)]
            dp = ixp_v[pl.ds(o, 16)]
            y = lb_v[pl.ds(o, 16)]
            egat = iotak + (o * K)
            m_end = (d != dn) | lane15
            m_start = (d != dp) | lane0
            mf = jnp.where(d == dn, one16, zero16)

            for k in range(K):
                t = plsc.load_gather(th_v, [egat + k])
                u = jnp.exp(-jnp.abs(t))
                bce = (jnp.maximum(t, zero16) - t * y + _log1p_poly(u)) * mf
                c_in = plsc.cumsum(bce)
                x_ex = bce - c_in
                dk = d + (k * B)
                plsc.addupdate_scatter(acc_v, [dk], c_in, mask=m_end)
                plsc.addupdate_scatter(acc_v, [dk], x_ex, mask=m_start)

                a = plsc.load_gather(al_v, [egat + k])
                ca = plsc.cumsum(a)
                xa = a - ca
                dka = d + ((K + k) * B)
                plsc.addupdate_scatter(acc_v, [dka], ca, mask=m_end)
                plsc.addupdate_scatter(acc_v, [dka], xa, mask=m_start)

            dc = d + (2 * K * B)
            plsc.addupdate_scatter(acc_v, [dc], cnt_end, mask=m_end)
            plsc.addupdate_scatter(acc_v, [dc], -cnt_start, mask=m_start)
            return _

        lax.fori_loop(0, NG, group_body, None)
        return _

    lax.fori_loop(0, NCH, chunk_body, None)
    pltpu.sync_copy(acc_v, out_h.at[wid])


def _tc_combine_kernel(p_ref, out_ref):
    S = jnp.sum(p_ref[...], axis=0)
    nll = S[0:K]
    A = S[K:2 * K]
    n = S[2 * K:2 * K + 1]
    ra = A / n
    ra_max = jnp.max(ra, axis=0, keepdims=True)
    ls = ra - ra_max - jnp.log(
        jnp.sum(jnp.exp(ra - ra_max), axis=0, keepdims=True))
    x = -nll + ls
    x_max = jnp.max(x, axis=0, keepdims=True)
    lp = x_max + jnp.log(jnp.sum(jnp.exp(x - x_max), axis=0,
                                 keepdims=True))
    loss_b = -lp / n
    out_ref[...] = jnp.sum(loss_b, axis=1, keepdims=True) / B


@jax.jit
def _run(label, log_theta, log_alpha, subgraph_idx):
    idx = subgraph_idx.astype(jnp.int32)
    idxn = jnp.concatenate([idx[1:], jnp.full((1,), B, jnp.int32)])
    idxp = jnp.concatenate([jnp.full((1,), -1, jnp.int32), idx[:-1]])

    mesh = plsc.VectorSubcoreMesh(core_axis_name="c", subcore_axis_name="s",
                                  num_cores=NC, num_subcores=NS)
    partials = pl.kernel(
        _sc_body,
        out_type=jax.ShapeDtypeStruct((NW, NACC * B), jnp.float32),
        mesh=mesh,
        compiler_params=pltpu.CompilerParams(needs_layout_passes=False),
        scratch_types=[
            pltpu.VMEM((NACC * B,), jnp.float32),
            pltpu.VMEM((CH * K,), jnp.float32),
            pltpu.VMEM((CH * K,), jnp.float32),
            pltpu.VMEM((CH,), jnp.float32),
            pltpu.VMEM((CH,), jnp.int32),
            pltpu.VMEM((CH,), jnp.int32),
            pltpu.VMEM((CH,), jnp.int32),
        ],
    )(label, log_theta.reshape(E * K), log_alpha.reshape(E * K),
      idx, idxn, idxp)

    out = pl.pallas_call(
        _tc_combine_kernel,
        out_shape=jax.ShapeDtypeStruct((1, 1), jnp.float32),
    )(partials.reshape(NW, NACC, B))
    return out[0, 0]


def kernel(label, log_theta, log_alpha, subgraph_idx, subgraph_idx_base,
           num_canonical_order):
    loss = _run(label, log_theta, log_alpha, subgraph_idx)
    return loss * jnp.asarray(num_canonical_order, jnp.float32)


# speedup vs baseline: 1.6983x; 1.6983x over previous
import functools

import jax
import jax.numpy as jnp
from jax import lax
from jax.experimental import pallas as pl
from jax.experimental.pallas import tpu as pltpu
from jax.experimental.pallas import tpu_sc as plsc

E = 1048576
K = 20
B = 2048
NC = 2
NS = 16
NW = NC * NS
EW = E // NW
CH = 512
NCH = EW // CH
NG = CH // 16
NACC = 2 * K + 1

_LOG1P = (2.554673020349618e-07, 0.9999670809438443, -0.49928504912226557,
          0.32722571497202635, -0.22316586411450423, 0.130833427976782,
          -0.05243753706207599, 0.01000928961639147)


def _log1p_poly(u):
    p = jnp.full((16,), _LOG1P[7], jnp.float32)
    for c in _LOG1P[6::-1]:
        p = p * u + c
    return p


def _sc_body(label_h, theta_h, alpha_h, idx_h, idxn_h, idxp_h, out_h,
             acc_v, th_v, al_v, lb_v, ix_v, ixn_v, ixp_v):
    cid = lax.axis_index("c")
    sid = lax.axis_index("s")
    wid = sid * NC + cid
    base = wid * EW

    zero16 = jnp.zeros((16,), jnp.float32)

    def zero_body(i, _):
        acc_v[pl.ds(i * 16, 16)] = zero16
        return _

    lax.fori_loop(0, NACC * B // 16, zero_body, None)

    iota = lax.iota(jnp.int32, 16)
    iotak = iota * K
    lane0 = iota == 0
    lane15 = iota == 15
    cnt_end = (iota + 1).astype(jnp.float32)
    cnt_start = iota.astype(jnp.float32)
    one16 = jnp.full((16,), 1.0, jnp.float32)

    def chunk_body(ci, _):
        e0 = base + ci * CH
        pltpu.sync_copy(theta_h.at[pl.ds(e0 * K, CH * K)], th_v)
        pltpu.sync_copy(alpha_h.at[pl.ds(e0 * K, CH * K)], al_v)
        pltpu.sync_copy(label_h.at[pl.ds(e0, CH)], lb_v)
        pltpu.sync_copy(idx_h.at[pl.ds(e0, CH)], ix_v)
        pltpu.sync_copy(idxn_h.at[pl.ds(e0, CH)], ixn_v)
        pltpu.sync_copy(idxp_h.at[pl.ds(e0, CH)], ixp_v)

        @plsc.parallel_loop(0, NG)
        def group_body(g):
            o = g * 16
            d = ix_v[pl.ds(o, 16)]
            dn = ixn_v[pl.ds(o, 16)]
            dp = ixp_v[pl.ds(o, 16)]
            y = lb_v[pl.ds(o, 16)]
            egat = iotak + (o * K)
            m_end = (d != dn) | lane15
            m_start = (d != dp) | lane0
            mf = jnp.where(d == dn, one16, zero16)

            for k in range(K):
                t = plsc.load_gather(th_v, [egat + k])
                u = jnp.exp(-jnp.abs(t))
                bce = (jnp.maximum(t, zero16) - t * y + _log1p_poly(u)) * mf
                c_in = plsc.cumsum(bce)
                x_ex = bce - c_in
                dk = d + (k * B)
                plsc.addupdate_scatter(acc_v, [dk], c_in, mask=m_end)
                plsc.addupdate_scatter(acc_v, [dk], x_ex, mask=m_start)

                a = plsc.load_gather(al_v, [egat + k])
                ca = plsc.cumsum(a)
                xa = a - ca
                dka = d + ((K + k) * B)
                plsc.addupdate_scatter(acc_v, [dka], ca, mask=m_end)
                plsc.addupdate_scatter(acc_v, [dka], xa, mask=m_start)

            dc = d + (2 * K * B)
            plsc.addupdate_scatter(acc_v, [dc], cnt_end, mask=m_end)
            plsc.addupdate_scatter(acc_v, [dc], -cnt_start, mask=m_start)

        return _

    lax.fori_loop(0, NCH, chunk_body, None)
    pltpu.sync_copy(acc_v, out_h.at[wid])


def _tc_combine_kernel(p_ref, out_ref):
    S = jnp.sum(p_ref[...], axis=0)
    nll = S[0:K]
    A = S[K:2 * K]
    n = S[2 * K:2 * K + 1]
    ra = A / n
    ra_max = jnp.max(ra, axis=0, keepdims=True)
    ls = ra - ra_max - jnp.log(
        jnp.sum(jnp.exp(ra - ra_max), axis=0, keepdims=True))
    x = -nll + ls
    x_max = jnp.max(x, axis=0, keepdims=True)
    lp = x_max + jnp.log(jnp.sum(jnp.exp(x - x_max), axis=0,
                                 keepdims=True))
    loss_b = -lp / n
    out_ref[...] = jnp.sum(loss_b, axis=1, keepdims=True) / B


@jax.jit
def _run(label, log_theta, log_alpha, subgraph_idx):
    idx = subgraph_idx.astype(jnp.int32)
    idxn = jnp.concatenate([idx[1:], jnp.full((1,), B, jnp.int32)])
    idxp = jnp.concatenate([jnp.full((1,), -1, jnp.int32), idx[:-1]])

    mesh = plsc.VectorSubcoreMesh(core_axis_name="c", subcore_axis_name="s",
                                  num_cores=NC, num_subcores=NS)
    partials = pl.kernel(
        _sc_body,
        out_type=jax.ShapeDtypeStruct((NW, NACC * B), jnp.float32),
        mesh=mesh,
        compiler_params=pltpu.CompilerParams(needs_layout_passes=False),
        scratch_types=[
            pltpu.VMEM((NACC * B,), jnp.float32),
            pltpu.VMEM((CH * K,), jnp.float32),
            pltpu.VMEM((CH * K,), jnp.float32),
            pltpu.VMEM((CH,), jnp.float32),
            pltpu.VMEM((CH,), jnp.int32),
            pltpu.VMEM((CH,), jnp.int32),
            pltpu.VMEM((CH,), jnp.int32),
        ],
    )(label, log_theta.reshape(E * K), log_alpha.reshape(E * K),
      idx, idxn, idxp)

    out = pl.pallas_call(
        _tc_combine_kernel,
        out_shape=jax.ShapeDtypeStruct((1, 1), jnp.float32),
    )(partials.reshape(NW, NACC, B))
    return out[0, 0]


def kernel(label, log_theta, log_alpha, subgraph_idx, subgraph_idx_base,
           num_canonical_order):
    loss = _run(label, log_theta, log_alpha, subgraph_idx)
    return loss * jnp.asarray(num_canonical_order, jnp.float32)


# speedup vs baseline: 2.1778x; 1.2823x over previous
import functools

import jax
import jax.numpy as jnp
from jax import lax
from jax.experimental import pallas as pl
from jax.experimental.pallas import tpu as pltpu
from jax.experimental.pallas import tpu_sc as plsc

E = 1048576
K = 20
B = 2048
NCOL = 2 * K
NACC = NCOL + 1
SUB = 128
NSUB = E // SUB
ROWS = NSUB * NCOL

EC2 = 8192
NSTEP = E // EC2
SC_PER_STEP = EC2 // SUB
ROWS_PER_STEP = SC_PER_STEP * NCOL

NC = 2
NS = 16
NW = NC * NS
EW = E // NW
CH = 512
CHS = CH // SUB
NCH = EW // CH
NG = CH // 16


def _pack_kernel(label_ref, theta_ref, alpha_ref, idx_ref, idxn_ref,
                 out_ref):
    theta = theta_ref[...]
    alpha = alpha_ref[...]
    lab = label_ref[...].reshape(EC2, 1)
    m = (idx_ref[...] == idxn_ref[...]).astype(jnp.float32).reshape(EC2, 1)
    bce = (jnp.maximum(theta, 0.0) - theta * lab
           + jnp.log1p(jnp.exp(-jnp.abs(theta)))) * m
    pack = jnp.concatenate([bce, alpha], axis=1)
    t = pack.reshape(SC_PER_STEP, SUB, NCOL).transpose(0, 2, 1)
    out_ref[...] = t.reshape(ROWS_PER_STEP, SUB)


def _sc_body(p_h, ixp_h, out_h, acc_v, pv, ix_v):
    cid = lax.axis_index("c")
    sid = lax.axis_index("s")
    wid = sid * NC + cid
    base = wid * EW

    zero16 = jnp.zeros((16,), jnp.float32)

    def zero_body(i, _):
        acc_v[pl.ds(i * 16, 16)] = zero16
        return _

    lax.fori_loop(0, NACC * B // 16, zero_body, None)

    iota = lax.iota(jnp.int32, 16)
    lane0 = iota == 0
    lane15 = iota == 15
    cnt_end = (iota + 1).astype(jnp.float32)
    cnt_start = iota.astype(jnp.float32)

    def chunk_body(ci, _):
        e0 = base + ci * CH
        row0 = (e0 // SUB) * NCOL
        pltpu.sync_copy(p_h.at[pl.ds(row0, CHS * NCOL)], pv)
        pltpu.sync_copy(ixp_h.at[pl.ds(e0, CH + 16)], ix_v)

        @plsc.parallel_loop(0, NG)
        def group_body(g):
            o = g * 16
            s = g // 8
            l = g % 8
            d = ix_v[pl.ds(8 + o, 16)]
            dn = ix_v[pl.ds(9 + o, 16)]
            dp = ix_v[pl.ds(7 + o, 16)]
            m_end = (d != dn) | lane15
            m_start = (d != dp) | lane0
            srow = s * NCOL
            lo = l * 16

            for col in range(NCOL):
                v = pv[srow + col, pl.ds(lo, 16)]
                c_in = plsc.cumsum(v)
                x_ex = v - c_in
                dk = d + (col * B)
                plsc.addupdate_scatter(acc_v, [dk], c_in, mask=m_end)
                plsc.addupdate_scatter(acc_v, [dk], x_ex, mask=m_start)

            dc = d + (NCOL * B)
            plsc.addupdate_scatter(acc_v, [dc], cnt_end, mask=m_end)
            plsc.addupdate_scatter(acc_v, [dc], -cnt_start, mask=m_start)

        return _

    lax.fori_loop(0, NCH, chunk_body, None)
    pltpu.sync_copy(acc_v, out_h.at[wid])


def _tc_combine_kernel(p_ref, out_ref):
    S = jnp.sum(p_ref[...], axis=0)
    nll = S[0:K]
    A = S[K:2 * K]
    n = S[2 * K:2 * K + 1]
    ra = A / n
    ra_max = jnp.max(ra, axis=0, keepdims=True)
    ls = ra - ra_max - jnp.log(
        jnp.sum(jnp.exp(ra - ra_max), axis=0, keepdims=True))
    x = -nll + ls
    x_max = jnp.max(x, axis=0, keepdims=True)
    lp = x_max + jnp.log(jnp.sum(jnp.exp(x - x_max), axis=0,
                                 keepdims=True))
    loss_b = -lp / n
    out_ref[...] = jnp.sum(loss_b, axis=1, keepdims=True) / B


@jax.jit
def _run(label, log_theta, log_alpha, subgraph_idx):
    idx = subgraph_idx.astype(jnp.int32)
    idxn = jnp.concatenate([idx[1:], jnp.full((1,), B, jnp.int32)])
    ixp = jnp.concatenate([jnp.full((8,), -1, jnp.int32), idx,
                           jnp.full((8,), B, jnp.int32)])

    packed = pl.pallas_call(
        _pack_kernel,
        grid=(NSTEP,),
        in_specs=[
            pl.BlockSpec((EC2,), lambda i: (i,)),
            pl.BlockSpec((EC2, K), lambda i: (i, 0)),
            pl.BlockSpec((EC2, K), lambda i: (i, 0)),
            pl.BlockSpec((EC2,), lambda i: (i,)),
            pl.BlockSpec((EC2,), lambda i: (i,)),
        ],
        out_specs=pl.BlockSpec((ROWS_PER_STEP, SUB), lambda i: (i, 0)),
        out_shape=jax.ShapeDtypeStruct((ROWS, SUB), jnp.float32),
    )(label, log_theta, log_alpha, idx, idxn)

    mesh = plsc.VectorSubcoreMesh(core_axis_name="c", subcore_axis_name="s",
                                  num_cores=NC, num_subcores=NS)
    partials = pl.kernel(
        _sc_body,
        out_type=jax.ShapeDtypeStruct((NW, NACC * B), jnp.float32),
        mesh=mesh,
        compiler_params=pltpu.CompilerParams(needs_layout_passes=False),
        scratch_types=[
            pltpu.VMEM((NACC * B,), jnp.float32),
            pltpu.VMEM((CHS * NCOL, SUB), jnp.float32),
            pltpu.VMEM((CH + 16,), jnp.int32),
        ],
    )(packed, ixp)

    out = pl.pallas_call(
        _tc_combine_kernel,
        out_shape=jax.ShapeDtypeStruct((1, 1), jnp.float32),
    )(partials.reshape(NW, NACC, B))
    return out[0, 0]


def kernel(label, log_theta, log_alpha, subgraph_idx, subgraph_idx_base,
           num_canonical_order):
    loss = _run(label, log_theta, log_alpha, subgraph_idx)
    return loss * jnp.asarray(num_canonical_order, jnp.float32)


# speedup vs baseline: 2.4244x; 1.1132x over previous
import functools

import jax
import jax.numpy as jnp
from jax import lax
from jax.experimental import pallas as pl
from jax.experimental.pallas import tpu as pltpu
from jax.experimental.pallas import tpu_sc as plsc

E = 1048576
K = 20
B = 2048
NCOL = 2 * K
NACC = NCOL + 1
SUB = 128

NHALF = 2
EH = E // NHALF
ROWS_H = EH // SUB * NCOL

EC2 = 8192
NSTEP_H = EH // EC2
SC_PER_STEP = EC2 // SUB
ROWS_PER_STEP = SC_PER_STEP * NCOL

NC = 2
NS = 16
NW = NC * NS
EW = EH // NW
CH = 512
CHS = CH // SUB
NCH = EW // CH
NG = CH // 16


def _pack_kernel(label_ref, theta_ref, alpha_ref, idx_ref, idxn_ref,
                 out_ref):
    theta = theta_ref[...]
    alpha = alpha_ref[...]
    lab = label_ref[...].reshape(EC2, 1)
    m = (idx_ref[...] == idxn_ref[...]).astype(jnp.float32).reshape(EC2, 1)
    bce = (jnp.maximum(theta, 0.0) - theta * lab
           + jnp.log1p(jnp.exp(-jnp.abs(theta)))) * m
    pack = jnp.concatenate([bce, alpha], axis=1)
    t = pack.reshape(SC_PER_STEP, SUB, NCOL).transpose(0, 2, 1)
    out_ref[...] = t.reshape(ROWS_PER_STEP, SUB)


def _make_sc_body(half):
    def _sc_body(p_h, ixp_h, out_h, acc_v, pv, ix_v):
        cid = lax.axis_index("c")
        sid = lax.axis_index("s")
        wid = sid * NC + cid
        base = wid * EW

        zero16 = jnp.zeros((16,), jnp.float32)

        def zero_body(i, _):
            acc_v[pl.ds(i * 16, 16)] = zero16
            return _

        lax.fori_loop(0, NACC * B // 16, zero_body, None)

        iota = lax.iota(jnp.int32, 16)
        lane0 = iota == 0
        lane15 = iota == 15
        cnt_end = (iota + 1).astype(jnp.float32)
        cnt_start = iota.astype(jnp.float32)

        def chunk_body(ci, _):
            e0 = base + ci * CH
            row0 = (e0 // SUB) * NCOL
            pltpu.sync_copy(p_h.at[pl.ds(row0, CHS * NCOL)], pv)
            pltpu.sync_copy(ixp_h.at[pl.ds(half * EH + e0, CH + 16)], ix_v)

            @plsc.parallel_loop(0, NG)
            def group_body(g):
                o = g * 16
                s = g // 8
                l = g % 8
                d = ix_v[pl.ds(8 + o, 16)]
                dn = ix_v[pl.ds(9 + o, 16)]
                dp = ix_v[pl.ds(7 + o, 16)]
                m_end = (d != dn) | lane15
                m_start = (d != dp) | lane0
                srow = s * NCOL
                lo = l * 16

                for col in range(NCOL):
                    v = pv[srow + col, pl.ds(lo, 16)]
                    c_in = plsc.cumsum(v)
                    x_ex = v - c_in
                    dk = d + (col * B)
                    plsc.addupdate_scatter(acc_v, [dk], c_in, mask=m_end)
                    plsc.addupdate_scatter(acc_v, [dk], x_ex, mask=m_start)

                dc = d + (NCOL * B)
                plsc.addupdate_scatter(acc_v, [dc], cnt_end, mask=m_end)
                plsc.addupdate_scatter(acc_v, [dc], -cnt_start,
                                       mask=m_start)

            return _

        lax.fori_loop(0, NCH, chunk_body, None)
        pltpu.sync_copy(acc_v, out_h.at[wid])

    return _sc_body


def _tc_combine_kernel(pa_ref, pb_ref, out_ref):
    S = jnp.sum(pa_ref[...], axis=0) + jnp.sum(pb_ref[...], axis=0)
    nll = S[0:K]
    A = S[K:2 * K]
    n = S[2 * K:2 * K + 1]
    ra = A / n
    ra_max = jnp.max(ra, axis=0, keepdims=True)
    ls = ra - ra_max - jnp.log(
        jnp.sum(jnp.exp(ra - ra_max), axis=0, keepdims=True))
    x = -nll + ls
    x_max = jnp.max(x, axis=0, keepdims=True)
    lp = x_max + jnp.log(jnp.sum(jnp.exp(x - x_max), axis=0,
                                 keepdims=True))
    loss_b = -lp / n
    out_ref[...] = jnp.sum(loss_b, axis=1, keepdims=True) / B


def _prepass(half, label, log_theta, log_alpha, idx, idxn):
    off = half * (EH // EC2)
    return pl.pallas_call(
        _pack_kernel,
        grid=(NSTEP_H,),
        in_specs=[
            pl.BlockSpec((EC2,), lambda i: (i + off,)),
            pl.BlockSpec((EC2, K), lambda i: (i + off, 0)),
            pl.BlockSpec((EC2, K), lambda i: (i + off, 0)),
            pl.BlockSpec((EC2,), lambda i: (i + off,)),
            pl.BlockSpec((EC2,), lambda i: (i + off,)),
        ],
        out_specs=pl.BlockSpec((ROWS_PER_STEP, SUB), lambda i: (i, 0)),
        out_shape=jax.ShapeDtypeStruct((ROWS_H, SUB), jnp.float32),
    )(label, log_theta, log_alpha, idx, idxn)


def _sc_reduce(half, packed, ixp):
    mesh = plsc.VectorSubcoreMesh(core_axis_name="c", subcore_axis_name="s",
                                  num_cores=NC, num_subcores=NS)
    return pl.kernel(
        _make_sc_body(half),
        out_type=jax.ShapeDtypeStruct((NW, NACC * B), jnp.float32),
        mesh=mesh,
        compiler_params=pltpu.CompilerParams(needs_layout_passes=False),
        scratch_types=[
            pltpu.VMEM((NACC * B,), jnp.float32),
            pltpu.VMEM((CHS * NCOL, SUB), jnp.float32),
            pltpu.VMEM((CH + 16,), jnp.int32),
        ],
    )(packed, ixp)


@jax.jit
def _run(label, log_theta, log_alpha, subgraph_idx):
    idx = subgraph_idx.astype(jnp.int32)
    idxn = jnp.concatenate([idx[1:], jnp.full((1,), B, jnp.int32)])
    ixp = jnp.concatenate([jnp.full((8,), -1, jnp.int32), idx,
                           jnp.full((8,), B, jnp.int32)])

    p0 = _prepass(0, label, log_theta, log_alpha, idx, idxn)
    parts0 = _sc_reduce(0, p0, ixp)
    p1 = _prepass(1, label, log_theta, log_alpha, idx, idxn)
    parts1 = _sc_reduce(1, p1, ixp)

    out = pl.pallas_call(
        _tc_combine_kernel,
        out_shape=jax.ShapeDtypeStruct((1, 1), jnp.float32),
    )(parts0.reshape(NW, NACC, B), parts1.reshape(NW, NACC, B))
    return out[0, 0]


def kernel(label, log_theta, log_alpha, subgraph_idx, subgraph_idx_base,
           num_canonical_order):
    loss = _run(label, log_theta, log_alpha, subgraph_idx)
    return loss * jnp.asarray(num_canonical_order, jnp.float32)


# speedup vs baseline: 2.5346x; 1.0455x over previous
import functools

import jax
import jax.numpy as jnp
from jax import lax
from jax.experimental import pallas as pl
from jax.experimental.pallas import tpu as pltpu
from jax.experimental.pallas import tpu_sc as plsc

E = 1048576
K = 20
B = 2048
NCOL = 2 * K
NACC = NCOL + 1
SUB = 128

NHALF = 4
EH = E // NHALF
ROWS_H = EH // SUB * NCOL

EC2 = 8192
NSTEP_H = EH // EC2
SC_PER_STEP = EC2 // SUB
ROWS_PER_STEP = SC_PER_STEP * NCOL

NC = 2
NS = 16
NW = NC * NS
EW = EH // NW
CH = 512
CHS = CH // SUB
NCH = EW // CH
NG = CH // 16


def _pack_kernel(label_ref, theta_ref, alpha_ref, idx_ref, idxn_ref,
                 out_ref):
    theta = theta_ref[...]
    alpha = alpha_ref[...]
    lab = label_ref[...].reshape(EC2, 1)
    m = (idx_ref[...] == idxn_ref[...]).astype(jnp.float32).reshape(EC2, 1)
    bce = (jnp.maximum(theta, 0.0) - theta * lab
           + jnp.log1p(jnp.exp(-jnp.abs(theta)))) * m
    pack = jnp.concatenate([bce, alpha], axis=1)
    t = pack.reshape(SC_PER_STEP, SUB, NCOL).transpose(0, 2, 1)
    out_ref[...] = t.reshape(ROWS_PER_STEP, SUB)


def _make_sc_body(half):
    def _sc_body(p_h, ixp_h, out_h, acc_v, pv, ix_v):
        cid = lax.axis_index("c")
        sid = lax.axis_index("s")
        wid = sid * NC + cid
        base = wid * EW

        zero16 = jnp.zeros((16,), jnp.float32)

        def zero_body(i, _):
            acc_v[pl.ds(i * 16, 16)] = zero16
            return _

        lax.fori_loop(0, NACC * B // 16, zero_body, None)

        iota = lax.iota(jnp.int32, 16)
        lane0 = iota == 0
        lane15 = iota == 15
        cnt_end = (iota + 1).astype(jnp.float32)
        cnt_start = iota.astype(jnp.float32)

        def chunk_body(ci, _):
            e0 = base + ci * CH
            row0 = (e0 // SUB) * NCOL
            pltpu.sync_copy(p_h.at[pl.ds(row0, CHS * NCOL)], pv)
            pltpu.sync_copy(ixp_h.at[pl.ds(half * EH + e0, CH + 16)], ix_v)

            @plsc.parallel_loop(0, NG)
            def group_body(g):
                o = g * 16
                s = g // 8
                l = g % 8
                d = ix_v[pl.ds(8 + o, 16)]
                dn = ix_v[pl.ds(9 + o, 16)]
                dp = ix_v[pl.ds(7 + o, 16)]
                m_end = (d != dn) | lane15
                m_start = (d != dp) | lane0
                srow = s * NCOL
                lo = l * 16

                for col in range(NCOL):
                    v = pv[srow + col, pl.ds(lo, 16)]
                    c_in = plsc.cumsum(v)
                    x_ex = v - c_in
                    dk = d + (col * B)
                    plsc.addupdate_scatter(acc_v, [dk], c_in, mask=m_end)
                    plsc.addupdate_scatter(acc_v, [dk], x_ex, mask=m_start)

                dc = d + (NCOL * B)
                plsc.addupdate_scatter(acc_v, [dc], cnt_end, mask=m_end)
                plsc.addupdate_scatter(acc_v, [dc], -cnt_start,
                                       mask=m_start)

            return _

        lax.fori_loop(0, NCH, chunk_body, None)
        pltpu.sync_copy(acc_v, out_h.at[wid])

    return _sc_body


def _tc_combine_kernel(*refs):
    out_ref = refs[-1]
    S = jnp.sum(refs[0][...], axis=0)
    for r in refs[1:-1]:
        S = S + jnp.sum(r[...], axis=0)
    nll = S[0:K]
    A = S[K:2 * K]
    n = S[2 * K:2 * K + 1]
    ra = A / n
    ra_max = jnp.max(ra, axis=0, keepdims=True)
    ls = ra - ra_max - jnp.log(
        jnp.sum(jnp.exp(ra - ra_max), axis=0, keepdims=True))
    x = -nll + ls
    x_max = jnp.max(x, axis=0, keepdims=True)
    lp = x_max + jnp.log(jnp.sum(jnp.exp(x - x_max), axis=0,
                                 keepdims=True))
    loss_b = -lp / n
    out_ref[...] = jnp.sum(loss_b, axis=1, keepdims=True) / B


def _prepass(half, label, log_theta, log_alpha, idx, idxn):
    off = half * (EH // EC2)
    return pl.pallas_call(
        _pack_kernel,
        grid=(NSTEP_H,),
        in_specs=[
            pl.BlockSpec((EC2,), lambda i: (i + off,)),
            pl.BlockSpec((EC2, K), lambda i: (i + off, 0)),
            pl.BlockSpec((EC2, K), lambda i: (i + off, 0)),
            pl.BlockSpec((EC2,), lambda i: (i + off,)),
            pl.BlockSpec((EC2,), lambda i: (i + off,)),
        ],
        out_specs=pl.BlockSpec((ROWS_PER_STEP, SUB), lambda i: (i, 0)),
        out_shape=jax.ShapeDtypeStruct((ROWS_H, SUB), jnp.float32),
    )(label, log_theta, log_alpha, idx, idxn)


def _sc_reduce(half, packed, ixp):
    mesh = plsc.VectorSubcoreMesh(core_axis_name="c", subcore_axis_name="s",
                                  num_cores=NC, num_subcores=NS)
    return pl.kernel(
        _make_sc_body(half),
        out_type=jax.ShapeDtypeStruct((NW, NACC * B), jnp.float32),
        mesh=mesh,
        compiler_params=pltpu.CompilerParams(needs_layout_passes=False),
        scratch_types=[
            pltpu.VMEM((NACC * B,), jnp.float32),
            pltpu.VMEM((CHS * NCOL, SUB), jnp.float32),
            pltpu.VMEM((CH + 16,), jnp.int32),
        ],
    )(packed, ixp)


@jax.jit
def _run(label, log_theta, log_alpha, subgraph_idx):
    idx = subgraph_idx.astype(jnp.int32)
    idxn = jnp.concatenate([idx[1:], jnp.full((1,), B, jnp.int32)])
    ixp = jnp.concatenate([jnp.full((8,), -1, jnp.int32), idx,
                           jnp.full((8,), B, jnp.int32)])

    parts = []
    for h in range(NHALF):
        p = _prepass(h, label, log_theta, log_alpha, idx, idxn)
        parts.append(_sc_reduce(h, p, ixp).reshape(NW, NACC, B))

    out = pl.pallas_call(
        _tc_combine_kernel,
        out_shape=jax.ShapeDtypeStruct((1, 1), jnp.float32),
    )(*parts)
    return out[0, 0]


def kernel(label, log_theta, log_alpha, subgraph_idx, subgraph_idx_base,
           num_canonical_order):
    loss = _run(label, log_theta, log_alpha, subgraph_idx)
    return loss * jnp.asarray(num_canonical_order, jnp.float32)
